# Initial kernel scaffold; baseline (speedup 1.0000x reference)
import jax
import jax.numpy as jnp
from jax.experimental import pallas as pl


def kernel(module_logits, task_ids):
    raise NotImplementedError("write your pallas kernel here")


# speedup vs baseline: 1.4919x; 1.4919x over previous
import functools

import jax
import jax.numpy as jnp
from jax.experimental import pallas as pl
from jax.experimental.pallas import tpu as pltpu
from jax.experimental.pallas import tpu_sc as plsc

N_TASKS = 1024
N_SPLITS = 8
N_SKILLS = 64
D = N_SPLITS * N_SKILLS
B = 16384
EPS = 1e-12

GATHER_WINDOW = 64


def _normalize_body(x_ref, o_ref):
    s = jax.nn.sigmoid(x_ref[...])
    o_ref[...] = s / (jnp.sum(s, axis=-1, keepdims=True) + EPS)


def _normalize_table(module_logits):
    x = module_logits.reshape(N_TASKS * N_SPLITS, N_SKILLS)
    out = pl.pallas_call(
        _normalize_body,
        out_shape=jax.ShapeDtypeStruct((N_TASKS * N_SPLITS, N_SKILLS), jnp.float32),
    )(x)
    return out.reshape(N_TASKS, D)


NC = 2
NS = 16
NW = NC * NS
B_PER_W = B // NW


def _sc_gather(table, idx):
    mesh = plsc.VectorSubcoreMesh(core_axis_name="c", subcore_axis_name="s")

    @functools.partial(
        pl.kernel,
        out_type=jax.ShapeDtypeStruct((B, D), jnp.float32),
        mesh=mesh,
        scratch_types=[
            pltpu.VMEM((B_PER_W,), jnp.int32),
            pltpu.VMEM((GATHER_WINDOW, D), jnp.float32),
            pltpu.SemaphoreType.DMA,
        ],
    )
    def k(table_hbm, idx_hbm, out_hbm, idx_v, rows_v, sem):
        wid = jax.lax.axis_index("s") * NC + jax.lax.axis_index("c")
        base = wid * B_PER_W
        pltpu.sync_copy(idx_hbm.at[pl.ds(base, B_PER_W)], idx_v)

        @pl.loop(0, B_PER_W, step=GATHER_WINDOW)
        def _(c):
            pltpu.async_copy(
                table_hbm.at[idx_v.at[pl.ds(c, GATHER_WINDOW)]], rows_v, sem
            ).wait()
            pltpu.sync_copy(rows_v, out_hbm.at[pl.ds(base + c, GATHER_WINDOW)])

    return k(table, idx)


def kernel(module_logits, task_ids):
    table = _normalize_table(module_logits)
    out = _sc_gather(table, task_ids.astype(jnp.int32))
    return out.reshape(B, N_SPLITS, N_SKILLS)


# speedup vs baseline: 1.5472x; 1.0370x over previous
import functools

import jax
import jax.numpy as jnp
from jax.experimental import pallas as pl
from jax.experimental.pallas import tpu as pltpu
from jax.experimental.pallas import tpu_sc as plsc

N_TASKS = 1024
N_SPLITS = 8
N_SKILLS = 64
D = N_SPLITS * N_SKILLS
B = 16384
EPS = 1e-12

GATHER_WINDOW = 64


def _normalize_body(x_ref, o_ref):
    s = jax.nn.sigmoid(x_ref[...])
    o_ref[...] = s / (jnp.sum(s, axis=-1, keepdims=True) + EPS)


def _normalize_table(module_logits):
    x = module_logits.reshape(N_TASKS * N_SPLITS, N_SKILLS)
    out = pl.pallas_call(
        _normalize_body,
        out_shape=jax.ShapeDtypeStruct((N_TASKS * N_SPLITS, N_SKILLS), jnp.float32),
    )(x)
    return out.reshape(N_TASKS, D)


NC = 2
NS = 16
NW = NC * NS
B_PER_W = B // NW


def _sc_gather(table, idx):
    mesh = plsc.VectorSubcoreMesh(core_axis_name="c", subcore_axis_name="s")

    @functools.partial(
        pl.kernel,
        out_type=jax.ShapeDtypeStruct((B, D), jnp.float32),
        mesh=mesh,
        scratch_types=[
            pltpu.VMEM((B_PER_W,), jnp.int32),
            pltpu.VMEM((2, GATHER_WINDOW, D), jnp.float32),
            pltpu.SemaphoreType.DMA((2,)),
            pltpu.SemaphoreType.DMA((2,)),
        ],
    )
    def k(table_hbm, idx_hbm, out_hbm, idx_v, rows_v, gsem, osem):
        wid = jax.lax.axis_index("s") * NC + jax.lax.axis_index("c")
        base = wid * B_PER_W
        pltpu.sync_copy(idx_hbm.at[pl.ds(base, B_PER_W)], idx_v)

        n = B_PER_W // GATHER_WINDOW
        W = GATHER_WINDOW
        g = [None] * n
        o = [None] * n
        for c in range(n):
            b = c % 2
            if c >= 2:
                o[c - 2].wait()
            g[c] = pltpu.async_copy(
                table_hbm.at[idx_v.at[pl.ds(c * W, W)]], rows_v.at[b], gsem.at[b]
            )
            if c >= 1:
                g[c - 1].wait()
                o[c - 1] = pltpu.async_copy(
                    rows_v.at[1 - b], out_hbm.at[pl.ds(base + (c - 1) * W, W)],
                    osem.at[1 - b],
                )
        g[n - 1].wait()
        o[n - 1] = pltpu.async_copy(
            rows_v.at[(n - 1) % 2], out_hbm.at[pl.ds(base + (n - 1) * W, W)],
            osem.at[(n - 1) % 2],
        )
        o[n - 2].wait()
        o[n - 1].wait()

    return k(table, idx)


def kernel(module_logits, task_ids):
    table = _normalize_table(module_logits)
    out = _sc_gather(table, task_ids.astype(jnp.int32))
    return out.reshape(B, N_SPLITS, N_SKILLS)


# speedup vs baseline: 1.7616x; 1.1386x over previous
import functools

import jax
import jax.numpy as jnp
from jax.experimental import pallas as pl
from jax.experimental.pallas import tpu as pltpu
from jax.experimental.pallas import tpu_sc as plsc

N_TASKS = 1024
N_SPLITS = 8
N_SKILLS = 64
D = N_SPLITS * N_SKILLS
B = 16384
EPS = 1e-12

GATHER_WINDOW = 64

NC = 2
NS = 16
NW = NC * NS


def _normalize_body(x_ref, o_ref):
    s = jax.nn.sigmoid(x_ref[...])
    for g in range(N_SPLITS):
        blk = s[:, g * N_SKILLS:(g + 1) * N_SKILLS]
        denom = jnp.sum(blk, axis=-1, keepdims=True) + EPS
        o_ref[:, g * N_SKILLS:(g + 1) * N_SKILLS] = blk / denom


def _normalize_table(module_logits):
    return pl.pallas_call(
        _normalize_body,
        out_shape=jax.ShapeDtypeStruct((N_TASKS, D), jnp.float32),
    )(module_logits)


def _sc_gather(table, idx, batch):
    mesh = plsc.VectorSubcoreMesh(core_axis_name="c", subcore_axis_name="s")
    b_per_w = batch // NW

    @functools.partial(
        pl.kernel,
        out_type=jax.ShapeDtypeStruct((batch, D), jnp.float32),
        mesh=mesh,
        scratch_types=[
            pltpu.VMEM((b_per_w,), jnp.int32),
            pltpu.VMEM((2, GATHER_WINDOW, D), jnp.float32),
            pltpu.SemaphoreType.DMA((2,)),
            pltpu.SemaphoreType.DMA((2,)),
        ],
    )
    def k(table_hbm, idx_hbm, out_hbm, idx_v, rows_v, gsem, osem):
        wid = jax.lax.axis_index("s") * NC + jax.lax.axis_index("c")
        base = wid * b_per_w
        pltpu.sync_copy(idx_hbm.at[pl.ds(base, b_per_w)], idx_v)

        n = b_per_w // GATHER_WINDOW
        W = GATHER_WINDOW
        g = [None] * n
        o = [None] * n
        for c in range(n):
            b = c % 2
            if c >= 2:
                o[c - 2].wait()
            g[c] = pltpu.async_copy(
                table_hbm.at[idx_v.at[pl.ds(c * W, W)]], rows_v.at[b], gsem.at[b]
            )
            if c >= 1:
                g[c - 1].wait()
                o[c - 1] = pltpu.async_copy(
                    rows_v.at[1 - b], out_hbm.at[pl.ds(base + (c - 1) * W, W)],
                    osem.at[1 - b],
                )
        g[n - 1].wait()
        o[n - 1] = pltpu.async_copy(
            rows_v.at[(n - 1) % 2], out_hbm.at[pl.ds(base + (n - 1) * W, W)],
            osem.at[(n - 1) % 2],
        )
        o[n - 2].wait()
        o[n - 1].wait()

    return k(table, idx)


def kernel(module_logits, task_ids):
    table = _normalize_table(module_logits)
    flat = _sc_gather(table, task_ids.astype(jnp.int32), B)
    return flat.reshape(B, N_SPLITS, N_SKILLS)
